# baseline (speedup 1.0000x reference)
import functools

import jax
import jax.numpy as jnp
from jax import lax
from jax.experimental import pallas as pl
from jax.experimental.pallas import tpu as pltpu
from jax.experimental.pallas import tpu_sc as plsc

N_NODES = 10000
N_PAD = 10240
N_EDGES = 320000
HIDDEN = 128
EDGE_IN = 16

NODE_BLK = 1280
EDGE_BLK = 3200

SC_CHUNK = 3200
SC_WINDOWS = SC_CHUNK // 16
SC_NCHUNKS = N_EDGES // SC_CHUNK
FSLICE = 4


def _encode_body(x_ref, we_ref, be_ref, wm1_ref, h_ref, gt_ref):
    h = jnp.maximum(x_ref[...] @ we_ref[...].T + be_ref[...], 0.0)
    h_ref[...] = h
    gt_ref[...] = wm1_ref[...] @ h.T


def _encode(x, We, be, Wm1):
    grid = (N_PAD // NODE_BLK,)
    return pl.pallas_call(
        _encode_body,
        grid=grid,
        in_specs=[
            pl.BlockSpec((NODE_BLK, HIDDEN), lambda i: (i, 0)),
            pl.BlockSpec((HIDDEN, HIDDEN), lambda i: (0, 0)),
            pl.BlockSpec((1, HIDDEN), lambda i: (0, 0)),
            pl.BlockSpec((HIDDEN, HIDDEN), lambda i: (0, 0)),
        ],
        out_specs=[
            pl.BlockSpec((NODE_BLK, HIDDEN), lambda i: (i, 0)),
            pl.BlockSpec((HIDDEN, NODE_BLK), lambda i: (0, i)),
        ],
        out_shape=[
            jax.ShapeDtypeStruct((N_PAD, HIDDEN), jnp.float32),
            jax.ShapeDtypeStruct((HIDDEN, N_PAD), jnp.float32),
        ],
    )(x, We, be.reshape(1, HIDDEN), Wm1)


def _edge_linear_body(ea_ref, wm2_ref, et_ref):
    et_ref[...] = wm2_ref[...] @ ea_ref[...].T


def _edge_linear(edge_attr, Wm2):
    grid = (N_EDGES // EDGE_BLK,)
    return pl.pallas_call(
        _edge_linear_body,
        grid=grid,
        in_specs=[
            pl.BlockSpec((EDGE_BLK, EDGE_IN), lambda i: (i, 0)),
            pl.BlockSpec((HIDDEN, EDGE_IN), lambda i: (0, 0)),
        ],
        out_specs=pl.BlockSpec((HIDDEN, EDGE_BLK), lambda i: (0, i)),
        out_shape=jax.ShapeDtypeStruct((HIDDEN, N_EDGES), jnp.float32),
    )(edge_attr, Wm2)


def _update_body(h_ref, at_ref, wu1_ref, wu2_ref, bm_ref, bu_ref, wd_ref,
                 bd_ref, o_ref):
    a = at_ref[...].T
    a = jnp.where(a == jnp.inf, 0.0, a + bm_ref[...])
    u = h_ref[...] @ wu1_ref[...].T + a @ wu2_ref[...].T + bu_ref[...]
    d = jnp.sum(u * wd_ref[...], axis=1, keepdims=True) + bd_ref[...]
    o_ref[...] = jax.nn.sigmoid(d)


def _update_decode(h, aggT, Wu1, Wu2, bm, bu, Wd, bd):
    grid = (N_PAD // NODE_BLK,)
    return pl.pallas_call(
        _update_body,
        grid=grid,
        in_specs=[
            pl.BlockSpec((NODE_BLK, HIDDEN), lambda i: (i, 0)),
            pl.BlockSpec((HIDDEN, NODE_BLK), lambda i: (0, i)),
            pl.BlockSpec((HIDDEN, HIDDEN), lambda i: (0, 0)),
            pl.BlockSpec((HIDDEN, HIDDEN), lambda i: (0, 0)),
            pl.BlockSpec((1, HIDDEN), lambda i: (0, 0)),
            pl.BlockSpec((1, HIDDEN), lambda i: (0, 0)),
            pl.BlockSpec((1, HIDDEN), lambda i: (0, 0)),
            pl.BlockSpec((1, 1), lambda i: (0, 0)),
        ],
        out_specs=pl.BlockSpec((NODE_BLK, 1), lambda i: (i, 0)),
        out_shape=jax.ShapeDtypeStruct((N_PAD, 1), jnp.float32),
    )(h, aggT, Wu1, Wu2, bm.reshape(1, HIDDEN), bu.reshape(1, HIDDEN),
      Wd.reshape(1, HIDDEN), bd.reshape(1, 1))


def _sc_aggregate(src, dst, gT, eT):
    mesh = plsc.VectorSubcoreMesh(core_axis_name="c", subcore_axis_name="s")

    @functools.partial(
        pl.kernel,
        mesh=mesh,
        compiler_params=pltpu.CompilerParams(needs_layout_passes=False),
        out_type=jax.ShapeDtypeStruct((HIDDEN, N_PAD), jnp.float32),
        scratch_types=[
            pltpu.VMEM((N_PAD,), jnp.float32),
            pltpu.VMEM((N_PAD,), jnp.float32),
            pltpu.VMEM((N_PAD,), jnp.float32),
            pltpu.VMEM((N_PAD,), jnp.float32),
            pltpu.VMEM((N_PAD,), jnp.float32),
            pltpu.VMEM((N_PAD,), jnp.float32),
            pltpu.VMEM((N_PAD,), jnp.float32),
            pltpu.VMEM((N_PAD,), jnp.float32),
            pltpu.VMEM((2 * SC_CHUNK,), jnp.int32),
            pltpu.VMEM((2 * SC_CHUNK,), jnp.int32),
            pltpu.VMEM((2 * FSLICE * SC_CHUNK,), jnp.float32),
            pltpu.VMEM((SC_WINDOWS // 8 * 16,), jnp.int32),
            pltpu.VMEM((N_PAD,), jnp.int32),
            pltpu.SemaphoreType.DMA,
            pltpu.SemaphoreType.DMA,
            pltpu.SemaphoreType.DMA,
            pltpu.SemaphoreType.DMA,
            pltpu.SemaphoreType.DMA,
            pltpu.SemaphoreType.DMA,
        ],
    )
    def agg(src_hbm, dst_hbm, gt_hbm, et_hbm, out_hbm, g0, g1, g2, g3,
            a0, a1, a2, a3, s_v, d_v, e_v, fl_v, tmp_v, *sems):
        g_refs = [g0, g1, g2, g3]
        a_refs = [a0, a1, a2, a3]
        w = lax.axis_index("s") * 2 + lax.axis_index("c")
        frow = w * FSLICE

        iota = lax.iota(jnp.int32, 16)
        inf16 = jnp.full((16,), jnp.inf, jnp.float32)
        lane_sel = [iota == r for r in range(16)]
        shift_idx = (iota + 1) & 15

        def fire(chunk, slot):
            base = chunk * SC_CHUNK
            pltpu.async_copy(src_hbm.at[pl.ds(base, SC_CHUNK)],
                             s_v.at[pl.ds(slot * SC_CHUNK, SC_CHUNK)],
                             sems[3 * slot])
            pltpu.async_copy(dst_hbm.at[pl.ds(base, SC_CHUNK)],
                             d_v.at[pl.ds(slot * SC_CHUNK, SC_CHUNK)],
                             sems[3 * slot + 1])
            for r in range(FSLICE):
                pltpu.async_copy(
                    et_hbm.at[frow + r, pl.ds(base, SC_CHUNK)],
                    e_v.at[pl.ds((2 * r + slot) * SC_CHUNK, SC_CHUNK)],
                    sems[3 * slot + 2])

        def drain(slot):
            pltpu.make_async_copy(
                src_hbm.at[pl.ds(0, SC_CHUNK)],
                s_v.at[pl.ds(slot * SC_CHUNK, SC_CHUNK)],
                sems[3 * slot]).wait()
            pltpu.make_async_copy(
                dst_hbm.at[pl.ds(0, SC_CHUNK)],
                d_v.at[pl.ds(slot * SC_CHUNK, SC_CHUNK)],
                sems[3 * slot + 1]).wait()
            for r in range(FSLICE):
                pltpu.make_async_copy(
                    et_hbm.at[frow + r, pl.ds(0, SC_CHUNK)],
                    e_v.at[pl.ds((2 * r + slot) * SC_CHUNK, SC_CHUNK)],
                    sems[3 * slot + 2]).wait()

        for r in range(FSLICE):
            pltpu.sync_copy(gt_hbm.at[frow + r, :], g_refs[r])

        def init_body(i, _):
            for r in range(FSLICE):
                a_refs[r][pl.ds(i * 16, 16)] = inf16
            return 0

        lax.fori_loop(0, N_PAD // 16, init_body, 0)

        fire(0, 0)
        fire(1, 1)

        def detect(slot, tb):
            d16 = d_v[pl.ds(slot * SC_CHUNK + tb, 16)]
            srt, _u = plsc.sort_key_val(d16, d16)
            sh = lax.gather(
                srt, shift_idx[:, None],
                lax.GatherDimensionNumbers(offset_dims=(),
                                           collapsed_slice_dims=(0,),
                                           start_index_map=(0,)),
                slice_sizes=(1,),
                mode=lax.GatherScatterMode.PROMISE_IN_BOUNDS)
            return jnp.max((srt == sh).astype(jnp.int32)), d16

        def messages(slot, tb, s16):
            ms = []
            for f in range(FSLICE):
                ev = e_v[pl.ds((2 * f + slot) * SC_CHUNK + tb, 16)]
                gv = plsc.load_gather(g_refs[f], [s16])
                ms.append(gv + ev)
            return ms

        def fast_update(slot, tb):
            d16 = d_v[pl.ds(slot * SC_CHUNK + tb, 16)]
            s16 = s_v[pl.ds(slot * SC_CHUNK + tb, 16)]
            plsc.store_scatter(tmp_v, [d16], iota)
            back = plsc.load_gather(tmp_v, [d16])
            ms = messages(slot, tb, s16)
            for f in range(FSLICE):
                av = plsc.load_gather(a_refs[f], [d16])
                plsc.store_scatter(a_refs[f], [d16], jnp.minimum(av, ms[f]))
            return (back != iota).astype(jnp.int32)

        def serial_update(slot, tb, d16):
            s16 = s_v[pl.ds(slot * SC_CHUNK + tb, 16)]
            ms = messages(slot, tb, s16)
            for f in range(FSLICE):
                for r in range(16):
                    av = plsc.load_gather(a_refs[f], [d16])
                    plsc.store_scatter(a_refs[f], [d16],
                                       jnp.minimum(av, ms[f]),
                                       mask=lane_sel[r])

        def super_body(k0, _):
            for b in range(2):
                drain(b)

                def win_body(t8, _):
                    orf = jnp.zeros((16,), jnp.int32)
                    for w in range(8):
                        orf = orf | fast_update(b, t8 * 128 + w * 16)
                    fl_v[pl.ds(t8 * 16, 16)] = orf
                    return 0

                lax.fori_loop(0, SC_WINDOWS // 8, win_body, 0)

                def fix_body(g, _):
                    def dirty():
                        for w in range(8):
                            tb = g * 128 + w * 16
                            p, d16 = detect(b, tb)
                            lax.cond(p > 0,
                                     lambda tb=tb, d16=d16:
                                         serial_update(b, tb, d16),
                                     lambda: None)

                    fl = fl_v[pl.ds(g * 16, 16)]
                    lax.cond(jnp.max(fl) > 0, dirty, lambda: None)
                    return 0

                lax.fori_loop(0, SC_WINDOWS // 8, fix_body, 0)

                def refire():
                    fire(2 * k0 + b + 2, b)

                lax.cond(2 * k0 + b + 2 < SC_NCHUNKS, refire, lambda: None)
            return 0

        lax.fori_loop(0, SC_NCHUNKS // 2, super_body, 0)

        for r in range(FSLICE):
            pltpu.sync_copy(a_refs[r], out_hbm.at[frow + r, :])

    return agg(src, dst, gT, eT)


def kernel(x, edge_index, edge_attr, We, be, Wm, bm, Wu, bu, Wd, bd):
    src = edge_index[0].astype(jnp.int32)
    dst = edge_index[1].astype(jnp.int32)
    Wm1 = Wm[:, :HIDDEN]
    Wm2 = Wm[:, HIDDEN:]
    Wu1 = Wu[:, :HIDDEN]
    Wu2 = Wu[:, HIDDEN:]

    x_pad = jnp.pad(x, ((0, N_PAD - N_NODES), (0, 0)))
    h, gT = _encode(x_pad, We, be, Wm1)
    eT = _edge_linear(edge_attr, Wm2)
    aggT = _sc_aggregate(src, dst, gT, eT)
    out = _update_decode(h, aggT, Wu1, Wu2, bm, bu, Wd, bd)
    return out[:N_NODES]


# speedup vs baseline: 1.0311x; 1.0311x over previous
import functools

import jax
import jax.numpy as jnp
from jax import lax
from jax.experimental import pallas as pl
from jax.experimental.pallas import tpu as pltpu
from jax.experimental.pallas import tpu_sc as plsc

N_NODES = 10000
N_PAD = 10240
N_EDGES = 320000
HIDDEN = 128
EDGE_IN = 16

NODE_BLK = 1280
EDGE_BLK = 3200

SC_CHUNK = 3200
SC_WINDOWS = SC_CHUNK // 16
SC_NCHUNKS = N_EDGES // SC_CHUNK
FSLICE = 4


def _encode_body(x_ref, we_ref, be_ref, wm1_ref, h_ref, gt_ref):
    h = jnp.maximum(x_ref[...] @ we_ref[...].T + be_ref[...], 0.0)
    h_ref[...] = h
    gt_ref[...] = wm1_ref[...] @ h.T


def _encode(x, We, be, Wm1):
    grid = (N_PAD // NODE_BLK,)
    return pl.pallas_call(
        _encode_body,
        grid=grid,
        in_specs=[
            pl.BlockSpec((NODE_BLK, HIDDEN), lambda i: (i, 0)),
            pl.BlockSpec((HIDDEN, HIDDEN), lambda i: (0, 0)),
            pl.BlockSpec((1, HIDDEN), lambda i: (0, 0)),
            pl.BlockSpec((HIDDEN, HIDDEN), lambda i: (0, 0)),
        ],
        out_specs=[
            pl.BlockSpec((NODE_BLK, HIDDEN), lambda i: (i, 0)),
            pl.BlockSpec((HIDDEN, NODE_BLK), lambda i: (0, i)),
        ],
        out_shape=[
            jax.ShapeDtypeStruct((N_PAD, HIDDEN), jnp.float32),
            jax.ShapeDtypeStruct((HIDDEN, N_PAD), jnp.float32),
        ],
    )(x, We, be.reshape(1, HIDDEN), Wm1)


def _edge_linear_body(ea_ref, wm2_ref, et_ref):
    et_ref[...] = wm2_ref[...] @ ea_ref[...].T


def _edge_linear(edge_attr, Wm2):
    grid = (N_EDGES // EDGE_BLK,)
    return pl.pallas_call(
        _edge_linear_body,
        grid=grid,
        in_specs=[
            pl.BlockSpec((EDGE_BLK, EDGE_IN), lambda i: (i, 0)),
            pl.BlockSpec((HIDDEN, EDGE_IN), lambda i: (0, 0)),
        ],
        out_specs=pl.BlockSpec((HIDDEN, EDGE_BLK), lambda i: (0, i)),
        out_shape=jax.ShapeDtypeStruct((HIDDEN, N_EDGES), jnp.float32),
    )(edge_attr, Wm2)


def _update_body(h_ref, at_ref, wu1_ref, wu2_ref, bm_ref, bu_ref, wd_ref,
                 bd_ref, o_ref):
    a = at_ref[...].T
    a = jnp.where(a == jnp.inf, 0.0, a + bm_ref[...])
    u = h_ref[...] @ wu1_ref[...].T + a @ wu2_ref[...].T + bu_ref[...]
    d = jnp.sum(u * wd_ref[...], axis=1, keepdims=True) + bd_ref[...]
    o_ref[...] = jax.nn.sigmoid(d)


def _update_decode(h, aggT, Wu1, Wu2, bm, bu, Wd, bd):
    grid = (N_PAD // NODE_BLK,)
    return pl.pallas_call(
        _update_body,
        grid=grid,
        in_specs=[
            pl.BlockSpec((NODE_BLK, HIDDEN), lambda i: (i, 0)),
            pl.BlockSpec((HIDDEN, NODE_BLK), lambda i: (0, i)),
            pl.BlockSpec((HIDDEN, HIDDEN), lambda i: (0, 0)),
            pl.BlockSpec((HIDDEN, HIDDEN), lambda i: (0, 0)),
            pl.BlockSpec((1, HIDDEN), lambda i: (0, 0)),
            pl.BlockSpec((1, HIDDEN), lambda i: (0, 0)),
            pl.BlockSpec((1, HIDDEN), lambda i: (0, 0)),
            pl.BlockSpec((1, 1), lambda i: (0, 0)),
        ],
        out_specs=pl.BlockSpec((NODE_BLK, 1), lambda i: (i, 0)),
        out_shape=jax.ShapeDtypeStruct((N_PAD, 1), jnp.float32),
    )(h, aggT, Wu1, Wu2, bm.reshape(1, HIDDEN), bu.reshape(1, HIDDEN),
      Wd.reshape(1, HIDDEN), bd.reshape(1, 1))


def _sc_aggregate(src, dst, gT, eT):
    mesh = plsc.VectorSubcoreMesh(core_axis_name="c", subcore_axis_name="s")

    @functools.partial(
        pl.kernel,
        mesh=mesh,
        compiler_params=pltpu.CompilerParams(needs_layout_passes=False),
        out_type=jax.ShapeDtypeStruct((HIDDEN, N_PAD), jnp.float32),
        scratch_types=[
            pltpu.VMEM((N_PAD,), jnp.float32),
            pltpu.VMEM((N_PAD,), jnp.float32),
            pltpu.VMEM((N_PAD,), jnp.float32),
            pltpu.VMEM((N_PAD,), jnp.float32),
            pltpu.VMEM((N_PAD,), jnp.float32),
            pltpu.VMEM((N_PAD,), jnp.float32),
            pltpu.VMEM((N_PAD,), jnp.float32),
            pltpu.VMEM((N_PAD,), jnp.float32),
            pltpu.VMEM((2 * SC_CHUNK,), jnp.int32),
            pltpu.VMEM((2 * SC_CHUNK,), jnp.int32),
            pltpu.VMEM((2 * FSLICE * SC_CHUNK,), jnp.float32),
            pltpu.VMEM((SC_WINDOWS // 8 * 16,), jnp.int32),
            pltpu.SemaphoreType.DMA,
            pltpu.SemaphoreType.DMA,
            pltpu.SemaphoreType.DMA,
            pltpu.SemaphoreType.DMA,
            pltpu.SemaphoreType.DMA,
            pltpu.SemaphoreType.DMA,
        ],
    )
    def agg(src_hbm, dst_hbm, gt_hbm, et_hbm, out_hbm, g0, g1, g2, g3,
            a0, a1, a2, a3, s_v, d_v, e_v, fl_v, *sems):
        g_refs = [g0, g1, g2, g3]
        a_refs = [a0, a1, a2, a3]
        w = lax.axis_index("s") * 2 + lax.axis_index("c")
        frow = w * FSLICE

        iota = lax.iota(jnp.int32, 16)
        inf16 = jnp.full((16,), jnp.inf, jnp.float32)
        lane_sel = [iota == r for r in range(16)]
        shift_idx = (iota + 1) & 15

        def fire(chunk, slot):
            base = chunk * SC_CHUNK
            pltpu.async_copy(src_hbm.at[pl.ds(base, SC_CHUNK)],
                             s_v.at[pl.ds(slot * SC_CHUNK, SC_CHUNK)],
                             sems[3 * slot])
            pltpu.async_copy(dst_hbm.at[pl.ds(base, SC_CHUNK)],
                             d_v.at[pl.ds(slot * SC_CHUNK, SC_CHUNK)],
                             sems[3 * slot + 1])
            for r in range(FSLICE):
                pltpu.async_copy(
                    et_hbm.at[frow + r, pl.ds(base, SC_CHUNK)],
                    e_v.at[pl.ds((2 * r + slot) * SC_CHUNK, SC_CHUNK)],
                    sems[3 * slot + 2])

        def drain(slot):
            pltpu.make_async_copy(
                src_hbm.at[pl.ds(0, SC_CHUNK)],
                s_v.at[pl.ds(slot * SC_CHUNK, SC_CHUNK)],
                sems[3 * slot]).wait()
            pltpu.make_async_copy(
                dst_hbm.at[pl.ds(0, SC_CHUNK)],
                d_v.at[pl.ds(slot * SC_CHUNK, SC_CHUNK)],
                sems[3 * slot + 1]).wait()
            for r in range(FSLICE):
                pltpu.make_async_copy(
                    et_hbm.at[frow + r, pl.ds(0, SC_CHUNK)],
                    e_v.at[pl.ds((2 * r + slot) * SC_CHUNK, SC_CHUNK)],
                    sems[3 * slot + 2]).wait()

        for r in range(FSLICE):
            pltpu.sync_copy(gt_hbm.at[frow + r, :], g_refs[r])

        def init_body(i, _):
            for r in range(FSLICE):
                a_refs[r][pl.ds(i * 16, 16)] = inf16
            return 0

        lax.fori_loop(0, N_PAD // 16, init_body, 0)

        fire(0, 0)
        fire(1, 1)

        def detect(slot, tb):
            d16 = d_v[pl.ds(slot * SC_CHUNK + tb, 16)]
            srt, _u = plsc.sort_key_val(d16, d16)
            sh = lax.gather(
                srt, shift_idx[:, None],
                lax.GatherDimensionNumbers(offset_dims=(),
                                           collapsed_slice_dims=(0,),
                                           start_index_map=(0,)),
                slice_sizes=(1,),
                mode=lax.GatherScatterMode.PROMISE_IN_BOUNDS)
            return jnp.max((srt == sh).astype(jnp.int32)), d16

        def messages(slot, tb, s16):
            ms = []
            for f in range(FSLICE):
                ev = e_v[pl.ds((2 * f + slot) * SC_CHUNK + tb, 16)]
                gv = plsc.load_gather(g_refs[f], [s16])
                ms.append(gv + ev)
            return ms

        def fast_update(slot, tb):
            d16 = d_v[pl.ds(slot * SC_CHUNK + tb, 16)]
            s16 = s_v[pl.ds(slot * SC_CHUNK + tb, 16)]
            srt, _u = plsc.sort_key_val(d16, d16)
            sh = lax.gather(
                srt, shift_idx[:, None],
                lax.GatherDimensionNumbers(offset_dims=(),
                                           collapsed_slice_dims=(0,),
                                           start_index_map=(0,)),
                slice_sizes=(1,),
                mode=lax.GatherScatterMode.PROMISE_IN_BOUNDS)
            ms = messages(slot, tb, s16)
            for f in range(FSLICE):
                av = plsc.load_gather(a_refs[f], [d16])
                plsc.store_scatter(a_refs[f], [d16], jnp.minimum(av, ms[f]))
            return (srt == sh).astype(jnp.int32)

        def serial_update(slot, tb, d16):
            s16 = s_v[pl.ds(slot * SC_CHUNK + tb, 16)]
            ms = messages(slot, tb, s16)
            for f in range(FSLICE):
                for r in range(16):
                    av = plsc.load_gather(a_refs[f], [d16])
                    plsc.store_scatter(a_refs[f], [d16],
                                       jnp.minimum(av, ms[f]),
                                       mask=lane_sel[r])

        def super_body(k0, _):
            for b in range(2):
                drain(b)

                def win_body(t8, _):
                    orf = jnp.zeros((16,), jnp.int32)
                    for w in range(8):
                        orf = orf | fast_update(b, t8 * 128 + w * 16)
                    fl_v[pl.ds(t8 * 16, 16)] = orf
                    return 0

                lax.fori_loop(0, SC_WINDOWS // 8, win_body, 0)

                def fix_body(g, _):
                    def dirty():
                        for w in range(8):
                            tb = g * 128 + w * 16
                            p, d16 = detect(b, tb)
                            lax.cond(p > 0,
                                     lambda tb=tb, d16=d16:
                                         serial_update(b, tb, d16),
                                     lambda: None)

                    fl = fl_v[pl.ds(g * 16, 16)]
                    lax.cond(jnp.max(fl) > 0, dirty, lambda: None)
                    return 0

                lax.fori_loop(0, SC_WINDOWS // 8, fix_body, 0)

                def refire():
                    fire(2 * k0 + b + 2, b)

                lax.cond(2 * k0 + b + 2 < SC_NCHUNKS, refire, lambda: None)
            return 0

        lax.fori_loop(0, SC_NCHUNKS // 2, super_body, 0)

        for r in range(FSLICE):
            pltpu.sync_copy(a_refs[r], out_hbm.at[frow + r, :])

    return agg(src, dst, gT, eT)


def kernel(x, edge_index, edge_attr, We, be, Wm, bm, Wu, bu, Wd, bd):
    src = edge_index[0].astype(jnp.int32)
    dst = edge_index[1].astype(jnp.int32)
    Wm1 = Wm[:, :HIDDEN]
    Wm2 = Wm[:, HIDDEN:]
    Wu1 = Wu[:, :HIDDEN]
    Wu2 = Wu[:, HIDDEN:]

    x_pad = jnp.pad(x, ((0, N_PAD - N_NODES), (0, 0)))
    h, gT = _encode(x_pad, We, be, Wm1)
    eT = _edge_linear(edge_attr, Wm2)
    aggT = _sc_aggregate(src, dst, gT, eT)
    out = _update_decode(h, aggT, Wu1, Wu2, bm, bu, Wd, bd)
    return out[:N_NODES]
